# Initial kernel scaffold; baseline (speedup 1.0000x reference)
import jax
import jax.numpy as jnp
from jax.experimental import pallas as pl


def kernel(x, lens, W, b, sos, label_seqs):
    raise NotImplementedError("write your pallas kernel here")


# speedup vs baseline: 585.2439x; 585.2439x over previous
import functools

import jax
import jax.numpy as jnp
from jax import lax
from jax.experimental import pallas as pl
from jax.experimental.pallas import tpu as pltpu


def _table_kernel(w_ref, b_ref, sos_ref, lab_ref, labt_ref, arg_ref, pred_ref,
                  *, C, T_l, J):
    F = T_l * J
    W = w_ref[...]
    b = b_ref[...]
    sos = sos_ref[...]
    lab = lab_ref[...]
    labt = labt_ref[...]
    cidx1 = lax.broadcasted_iota(jnp.int32, (1, C), 1)
    cidxC = lax.broadcasted_iota(jnp.int32, (C, C), 1)

    pred_table = jnp.concatenate(
        [jnp.dot(lab[:, t * J:(t + 1) * J], W,
                 preferred_element_type=jnp.float32) + b
         for t in range(T_l)], axis=1)

    lnorms = []
    acc = jnp.zeros((1, C), jnp.float32)
    for t in range(T_l):
        sl = labt[t * J:(t + 1) * J, :]
        acc = acc + jnp.sum(sl * sl, axis=0, keepdims=True)
        lnorms.append(acc)

    def classify(pd, t, cidx):
        score = lnorms[t - 1] - 2.0 * pd
        minv = jnp.min(score, axis=-1, keepdims=True)
        return jnp.min(jnp.where(score == minv, cidx, C),
                       axis=-1, keepdims=True)

    onehots = []
    for t in range(1, T_l + 1):
        pd = jnp.dot(pred_table[:, :t * J], labt[:t * J, :],
                     preferred_element_type=jnp.float32)
        arg = classify(pd, t, cidxC)
        onehots.append((cidxC == arg).astype(jnp.float32))

    p0 = jnp.dot(sos, W, preferred_element_type=jnp.float32) + b
    pd0 = jnp.dot(p0, labt[:J, :], preferred_element_type=jnp.float32)
    arg0 = classify(pd0, 1, cidx1)
    oh = (cidx1 == arg0).astype(jnp.float32)

    for s in range(1, T_l):
        oh = jnp.dot(oh, onehots[s - 1], preferred_element_type=jnp.float32)

    pred_fin = jnp.dot(oh, pred_table, preferred_element_type=jnp.float32)
    oh_fin = jnp.dot(oh, onehots[T_l - 1], preferred_element_type=jnp.float32)
    arg_fin = jnp.min(jnp.where(oh_fin > 0.5, cidx1, C),
                      axis=-1, keepdims=True)
    arg_ref[...] = jnp.broadcast_to(arg_fin, arg_ref.shape)
    pred_ref[...] = pred_fin


def kernel(x, lens, W, b, sos, label_seqs):
    B = x.shape[0]
    C, T_l, J = label_seqs.shape
    F = T_l * J

    lab = label_seqs.astype(jnp.float32).reshape(C, F)
    labt = lab.T
    b2 = b.astype(jnp.float32).reshape(1, J)
    sos2 = sos.astype(jnp.float32).reshape(1, J)
    Wf = W.astype(jnp.float32)

    kern = functools.partial(_table_kernel, C=C, T_l=T_l, J=J)
    arg_out, pred_out = pl.pallas_call(
        kern,
        out_shape=(jax.ShapeDtypeStruct((1, C), jnp.int32),
                   jax.ShapeDtypeStruct((1, F), jnp.float32)),
        grid=(1,),
        in_specs=[
            pl.BlockSpec((J, J), lambda i: (0, 0)),
            pl.BlockSpec((1, J), lambda i: (0, 0)),
            pl.BlockSpec((1, J), lambda i: (0, 0)),
            pl.BlockSpec((C, F), lambda i: (0, 0)),
            pl.BlockSpec((F, C), lambda i: (0, 0)),
        ],
        out_specs=(pl.BlockSpec((1, C), lambda i: (0, 0)),
                   pl.BlockSpec((1, F), lambda i: (0, 0))),
        compiler_params=pltpu.CompilerParams(
            dimension_semantics=("arbitrary",)),
    )(Wf, b2, sos2, lab, labt)

    pred_label_sofar = jnp.broadcast_to(arg_out[0, 0], (B,))
    pred_label_seq = jnp.broadcast_to(pred_out.reshape(1, T_l, J), (B, T_l, J))
    return pred_label_sofar, pred_label_seq
